# baseline (speedup 1.0000x reference)
import functools

import jax
import jax.numpy as jnp
from jax import lax
from jax.experimental import pallas as pl
from jax.experimental.pallas import tpu as pltpu
from jax.experimental.pallas import tpu_sc as plsc

E = 16
K = 2
D = 1024
F = 2048
T = 2048
C = 320
CP = 328
NB = E * CP
TRASH = NB
BR = NB + 8

SW = 128
NTILES = 32
TPT = T // NTILES

EH = E // 2
HY = EH * CP


def _router_body(logits_ref, dsp0, dsp1,
                 cmb00, cmb01, cmb10, cmb11, wb0, wb1, s_ref):
    logits = logits_ref[...]
    lane = lax.broadcasted_iota(jnp.int32, (T, E), 1)

    m = jnp.max(logits, axis=1, keepdims=True)
    ex = jnp.exp(logits - m)
    p = ex / jnp.sum(ex, axis=1, keepdims=True)

    m1 = jnp.max(p, axis=1, keepdims=True)
    i1 = jnp.min(jnp.where(p == m1, lane, E), axis=1, keepdims=True)
    p2 = jnp.where(lane == i1, -1.0, p)
    m2 = jnp.max(p2, axis=1, keepdims=True)
    i2 = jnp.min(jnp.where(p2 == m2, lane, E), axis=1, keepdims=True)

    w1 = m1 / (m1 + m2)
    w2 = m2 / (m1 + m2)

    oh1 = (lane == i1)
    oh2 = (lane == i2)
    x = oh1.astype(jnp.float32) + oh2.astype(jnp.float32)

    def blk(b, _):
        row = b * 128 + lax.broadcasted_iota(jnp.int32, (128, T), 0)
        col = lax.broadcasted_iota(jnp.int32, (128, T), 1)
        mm = (col < row).astype(jnp.float32)
        s_ref[pl.ds(b * 128, 128), :] = jnp.dot(
            mm, x, preferred_element_type=jnp.float32)
        return 0
    lax.fori_loop(0, T // 128, blk, 0)
    s = s_ref[...]

    pos1 = jnp.sum(jnp.where(oh1, s, 0.0), axis=1, keepdims=True).astype(jnp.int32)
    pos2 = jnp.sum(jnp.where(oh2, s, 0.0), axis=1, keepdims=True).astype(jnp.int32)
    keep1 = pos1 < C
    keep2 = pos2 < C

    dsp0[...] = jnp.where(keep1, i1 * CP + pos1, TRASH)
    dsp1[...] = jnp.where(keep2, i2 * CP + pos2, TRASH)
    l1 = (i1 & 7) * CP + pos1
    l2 = (i2 & 7) * CP + pos2
    z1 = (i1 & 7) * CP + C
    z2 = (i2 & 7) * CP + C
    cmb00[...] = jnp.where(keep1 & (i1 < EH), l1, z1)
    cmb01[...] = jnp.where(keep2 & (i2 < EH), l2, z2)
    cmb10[...] = jnp.where(keep1 & (i1 >= EH), l1, z1)
    cmb11[...] = jnp.where(keep2 & (i2 >= EH), l2, z2)
    wb0[...] = jnp.broadcast_to(w1, (T, SW))
    wb1[...] = jnp.broadcast_to(w2, (T, SW))


def _router(router_logits):
    i32 = jnp.int32
    return pl.pallas_call(
        _router_body,
        out_shape=(
            jax.ShapeDtypeStruct((T, 1), i32),
            jax.ShapeDtypeStruct((T, 1), i32),
            jax.ShapeDtypeStruct((T, 1), i32),
            jax.ShapeDtypeStruct((T, 1), i32),
            jax.ShapeDtypeStruct((T, 1), i32),
            jax.ShapeDtypeStruct((T, 1), i32),
            jax.ShapeDtypeStruct((T, SW), jnp.float32),
            jax.ShapeDtypeStruct((T, SW), jnp.float32),
        ),
        scratch_shapes=[pltpu.VMEM((T, E), jnp.float32)],
    )(router_logits)


def _dispatch_body(hid, dsp0, dsp1, wb0, wb1, buf, scale,
                   idx0_v, idx1_v, rows_v, wrow0_v, wrow1_v, zrow_v, sem):
    c = lax.axis_index("c")
    s = lax.axis_index("s")
    w = s * 2 + c
    t0 = w * TPT

    pltpu.sync_copy(dsp0.at[pl.ds(t0, TPT)], idx0_v)
    pltpu.sync_copy(dsp1.at[pl.ds(t0, TPT)], idx1_v)
    pltpu.sync_copy(wb0.at[pl.ds(t0, TPT)], wrow0_v)
    pltpu.sync_copy(wb1.at[pl.ds(t0, TPT)], wrow1_v)
    pltpu.sync_copy(hid.at[pl.ds(t0, TPT)], rows_v)
    d0 = pltpu.async_copy(rows_v, buf.at[idx0_v], sem)
    d1 = pltpu.async_copy(rows_v, buf.at[idx1_v], sem)
    d2 = pltpu.async_copy(wrow0_v, scale.at[idx0_v], sem)
    d3 = pltpu.async_copy(wrow1_v, scale.at[idx1_v], sem)

    def zv(i, _):
        zrow_v[0, pl.ds(i * 16, 16)] = jnp.zeros((16,), jnp.float32)
        return 0
    lax.fori_loop(0, D // 16, zv, 0)

    @pl.when(w < E)
    def _():
        zr = w * CP + C
        pltpu.sync_copy(zrow_v, buf.at[pl.ds(zr, 1)])
        pltpu.sync_copy(zrow_v.at[:, pl.ds(0, SW)], scale.at[pl.ds(zr, 1)])

    d0.wait()
    d1.wait()
    d2.wait()
    d3.wait()


def _dispatch(hidden_states, dsp0, dsp1, wb0, wb1):
    mesh = plsc.VectorSubcoreMesh(core_axis_name="c", subcore_axis_name="s")
    f32 = jnp.float32
    kfn = functools.partial(
        pl.kernel,
        out_type=(
            jax.ShapeDtypeStruct((BR, D), f32),
            jax.ShapeDtypeStruct((BR, SW), f32),
        ),
        mesh=mesh,
        scratch_types=[
            pltpu.VMEM((TPT,), jnp.int32),
            pltpu.VMEM((TPT,), jnp.int32),
            pltpu.VMEM((TPT, D), f32),
            pltpu.VMEM((TPT, SW), f32),
            pltpu.VMEM((TPT, SW), f32),
            pltpu.VMEM((1, D), f32),
            pltpu.SemaphoreType.DMA,
        ],
    )(_dispatch_body)
    return kfn(hidden_states, dsp0, dsp1, wb0, wb1)


_NF = 2
_FB = F // _NF


def _ffn_body(x_ref, w1_ref, w2_ref, sc_ref, y_ref):
    f = pl.program_id(1)

    @pl.when(f == 0)
    def _():
        y_ref[...] = jnp.zeros_like(y_ref)

    x = x_ref[...]
    h = jnp.dot(x, w1_ref[0], preferred_element_type=jnp.float32)
    c0 = 0.7978845608028654
    g = 0.5 * h * (1.0 + jnp.tanh(c0 * (h + 0.044715 * h * h * h)))
    y_ref[...] += jnp.dot(g, w2_ref[0], preferred_element_type=jnp.float32)

    @pl.when(f == _NF - 1)
    def _():
        y_ref[...] *= sc_ref[:, 0:1]


def _ffn(buf, scale, W1, W2, h):
    return pl.pallas_call(
        _ffn_body,
        grid=(EH, _NF),
        in_specs=[
            pl.BlockSpec((CP, D), lambda e, f: (e + h * EH, 0)),
            pl.BlockSpec((1, D, _FB), lambda e, f: (e + h * EH, 0, f)),
            pl.BlockSpec((1, _FB, D), lambda e, f: (e + h * EH, f, 0)),
            pl.BlockSpec((CP, SW), lambda e, f: (e + h * EH, 0)),
        ],
        out_specs=pl.BlockSpec((CP, D), lambda e, f: (e, 0)),
        out_shape=jax.ShapeDtypeStruct((HY, D), jnp.float32),
    )(buf, W1, W2, scale)


_CCH = 16


def _combine_half0_body(y, cmb0, cmb1, out,
                        idx0a, idx1a, idx0b, idx1b, r0a, r1a, r0b, r1b,
                        sema, semb):
    c = lax.axis_index("c")
    s = lax.axis_index("s")
    w = s * 2 + c
    t0 = w * TPT

    bufs = ((idx0a, idx1a, r0a, r1a, sema), (idx0b, idx1b, r0b, r1b, semb))
    nch = TPT // _CCH

    def issue(ch, bset):
        i0, i1, r0, r1, sem = bset
        base = t0 + ch * _CCH
        pltpu.sync_copy(cmb0.at[pl.ds(base, _CCH)], i0)
        pltpu.sync_copy(cmb1.at[pl.ds(base, _CCH)], i1)
        return (pltpu.async_copy(y.at[i0], r0, sem),
                pltpu.async_copy(y.at[i1], r1, sem))

    pend = issue(0, bufs[0])
    for ch in range(nch):
        cur = bufs[ch % 2]
        if ch + 1 < nch:
            nxt = issue(ch + 1, bufs[(ch + 1) % 2])
        for d in pend:
            d.wait()
        _, _, r0, r1, _ = cur

        def vadd(v, _):
            for tk in range(_CCH):
                r0[tk, pl.ds(v * 16, 16)] = (
                    r0[tk, pl.ds(v * 16, 16)] + r1[tk, pl.ds(v * 16, 16)])
            return 0
        lax.fori_loop(0, D // 16, vadd, 0)
        pltpu.sync_copy(r0, out.at[pl.ds(t0 + ch * _CCH, _CCH)])
        if ch + 1 < nch:
            pend = nxt


def _combine_half1_body(y, cmb0, cmb1, acc, out,
                        idx0a, idx1a, idx0b, idx1b,
                        r0a, r1a, r0b, r1b, oa, ob, sema, semb):
    c = lax.axis_index("c")
    s = lax.axis_index("s")
    w = s * 2 + c
    t0 = w * TPT

    bufs = ((idx0a, idx1a, r0a, r1a, oa, sema),
            (idx0b, idx1b, r0b, r1b, ob, semb))
    nch = TPT // _CCH

    def issue(ch, bset):
        i0, i1, r0, r1, o, sem = bset
        base = t0 + ch * _CCH
        pltpu.sync_copy(cmb0.at[pl.ds(base, _CCH)], i0)
        pltpu.sync_copy(cmb1.at[pl.ds(base, _CCH)], i1)
        return (pltpu.async_copy(y.at[i0], r0, sem),
                pltpu.async_copy(y.at[i1], r1, sem),
                pltpu.async_copy(acc.at[pl.ds(base, _CCH)], o, sem))

    pend = issue(0, bufs[0])
    for ch in range(nch):
        cur = bufs[ch % 2]
        if ch + 1 < nch:
            nxt = issue(ch + 1, bufs[(ch + 1) % 2])
        for d in pend:
            d.wait()
        _, _, r0, r1, o, _ = cur

        def vadd(v, _):
            for tk in range(_CCH):
                r0[tk, pl.ds(v * 16, 16)] = (
                    r0[tk, pl.ds(v * 16, 16)] + r1[tk, pl.ds(v * 16, 16)]
                    + o[tk, pl.ds(v * 16, 16)])
            return 0
        lax.fori_loop(0, D // 16, vadd, 0)
        pltpu.sync_copy(r0, out.at[pl.ds(t0 + ch * _CCH, _CCH)])
        if ch + 1 < nch:
            pend = nxt


def _combine(y0, cmb00, cmb01, y1, cmb10, cmb11):
    mesh = plsc.VectorSubcoreMesh(core_axis_name="c", subcore_axis_name="s")
    f32 = jnp.float32
    i32 = jnp.int32
    idx_t = [pltpu.VMEM((_CCH,), i32)] * 4
    row_t = [pltpu.VMEM((_CCH, D), f32)]
    sem_t = [pltpu.SemaphoreType.DMA] * 2
    out0 = functools.partial(
        pl.kernel,
        out_type=jax.ShapeDtypeStruct((T, D), f32),
        mesh=mesh,
        scratch_types=idx_t + row_t * 4 + sem_t,
    )(_combine_half0_body)(y0, cmb00, cmb01)
    return functools.partial(
        pl.kernel,
        out_type=jax.ShapeDtypeStruct((T, D), f32),
        mesh=mesh,
        scratch_types=idx_t + row_t * 6 + sem_t,
    )(_combine_half1_body)(y1, cmb10, cmb11, out0)


def kernel(hidden_states, router_logits, W1, W2):
    (dsp0, dsp1, cmb00, cmb01, cmb10, cmb11, wb0, wb1) = _router(router_logits)
    buf, scale = _dispatch(
        hidden_states, dsp0.reshape(T), dsp1.reshape(T), wb0, wb1)
    y0 = _ffn(buf, scale, W1, W2, 0)
    y1 = _ffn(buf, scale, W1, W2, 1)
    return _combine(y0, cmb00.reshape(T), cmb01.reshape(T),
                    y1, cmb10.reshape(T), cmb11.reshape(T))


# speedup vs baseline: 1.4595x; 1.4595x over previous
import functools

import jax
import jax.numpy as jnp
from jax import lax
from jax.experimental import pallas as pl
from jax.experimental.pallas import tpu as pltpu
from jax.experimental.pallas import tpu_sc as plsc

E = 16
K = 2
D = 1024
F = 2048
T = 2048
C = 320
CP = 328
NB = E * CP
TRASH = NB
BR = NB + 8

SW = 128
NTILES = 32
TPT = T // NTILES


def _router_body(logits_ref, dsp0, dsp1, cmb0, cmb1, wb0, wb1, s_ref):
    logits = logits_ref[...]
    lane = lax.broadcasted_iota(jnp.int32, (T, E), 1)

    m = jnp.max(logits, axis=1, keepdims=True)
    ex = jnp.exp(logits - m)
    p = ex / jnp.sum(ex, axis=1, keepdims=True)

    m1 = jnp.max(p, axis=1, keepdims=True)
    i1 = jnp.min(jnp.where(p == m1, lane, E), axis=1, keepdims=True)
    p2 = jnp.where(lane == i1, -1.0, p)
    m2 = jnp.max(p2, axis=1, keepdims=True)
    i2 = jnp.min(jnp.where(p2 == m2, lane, E), axis=1, keepdims=True)

    w1 = m1 / (m1 + m2)
    w2 = m2 / (m1 + m2)

    oh1 = (lane == i1)
    oh2 = (lane == i2)
    x = oh1.astype(jnp.float32) + oh2.astype(jnp.float32)

    def blk(b, _):
        row = b * 128 + lax.broadcasted_iota(jnp.int32, (128, T), 0)
        col = lax.broadcasted_iota(jnp.int32, (128, T), 1)
        mm = (col < row).astype(jnp.float32)
        s_ref[pl.ds(b * 128, 128), :] = jnp.dot(
            mm, x, preferred_element_type=jnp.float32)
        return 0
    lax.fori_loop(0, T // 128, blk, 0)
    s = s_ref[...]

    pos1 = jnp.sum(jnp.where(oh1, s, 0.0), axis=1, keepdims=True).astype(jnp.int32)
    pos2 = jnp.sum(jnp.where(oh2, s, 0.0), axis=1, keepdims=True).astype(jnp.int32)
    keep1 = pos1 < C
    keep2 = pos2 < C

    dst1 = i1 * CP + pos1
    dst2 = i2 * CP + pos2
    dsp0[...] = jnp.where(keep1, dst1, TRASH)
    dsp1[...] = jnp.where(keep2, dst2, TRASH)
    cmb0[...] = jnp.where(keep1, dst1, i1 * CP + C)
    cmb1[...] = jnp.where(keep2, dst2, i2 * CP + C)
    wb0[...] = jnp.broadcast_to(w1, (T, SW))
    wb1[...] = jnp.broadcast_to(w2, (T, SW))


def _router(router_logits):
    i32 = jnp.int32
    return pl.pallas_call(
        _router_body,
        out_shape=(
            jax.ShapeDtypeStruct((T, 1), i32),
            jax.ShapeDtypeStruct((T, 1), i32),
            jax.ShapeDtypeStruct((T, 1), i32),
            jax.ShapeDtypeStruct((T, 1), i32),
            jax.ShapeDtypeStruct((T, SW), jnp.float32),
            jax.ShapeDtypeStruct((T, SW), jnp.float32),
        ),
        scratch_shapes=[pltpu.VMEM((T, E), jnp.float32)],
    )(router_logits)


def _dispatch_body(hid, dsp0, dsp1, wb0, wb1, buf, scale,
                   idx0_v, idx1_v, rows_v, wrow0_v, wrow1_v, zrow_v, sem):
    c = lax.axis_index("c")
    s = lax.axis_index("s")
    w = s * 2 + c
    t0 = w * TPT

    pltpu.sync_copy(dsp0.at[pl.ds(t0, TPT)], idx0_v)
    pltpu.sync_copy(dsp1.at[pl.ds(t0, TPT)], idx1_v)
    pltpu.sync_copy(wb0.at[pl.ds(t0, TPT)], wrow0_v)
    pltpu.sync_copy(wb1.at[pl.ds(t0, TPT)], wrow1_v)
    pltpu.sync_copy(hid.at[pl.ds(t0, TPT)], rows_v)
    d0 = pltpu.async_copy(rows_v, buf.at[idx0_v], sem)
    d1 = pltpu.async_copy(rows_v, buf.at[idx1_v], sem)
    d2 = pltpu.async_copy(wrow0_v, scale.at[idx0_v], sem)
    d3 = pltpu.async_copy(wrow1_v, scale.at[idx1_v], sem)

    def zv(i, _):
        zrow_v[0, pl.ds(i * 16, 16)] = jnp.zeros((16,), jnp.float32)
        return 0
    lax.fori_loop(0, D // 16, zv, 0)

    @pl.when(w < E)
    def _():
        zr = w * CP + C
        pltpu.sync_copy(zrow_v, buf.at[pl.ds(zr, 1)])
        pltpu.sync_copy(zrow_v.at[:, pl.ds(0, SW)], scale.at[pl.ds(zr, 1)])

    d0.wait()
    d1.wait()
    d2.wait()
    d3.wait()


def _dispatch(hidden_states, dsp0, dsp1, wb0, wb1):
    mesh = plsc.VectorSubcoreMesh(core_axis_name="c", subcore_axis_name="s")
    f32 = jnp.float32
    kfn = functools.partial(
        pl.kernel,
        out_type=(
            jax.ShapeDtypeStruct((BR, D), f32),
            jax.ShapeDtypeStruct((BR, SW), f32),
        ),
        mesh=mesh,
        scratch_types=[
            pltpu.VMEM((TPT,), jnp.int32),
            pltpu.VMEM((TPT,), jnp.int32),
            pltpu.VMEM((TPT, D), f32),
            pltpu.VMEM((TPT, SW), f32),
            pltpu.VMEM((TPT, SW), f32),
            pltpu.VMEM((1, D), f32),
            pltpu.SemaphoreType.DMA,
        ],
    )(_dispatch_body)
    return kfn(hidden_states, dsp0, dsp1, wb0, wb1)


_NF = 1
_FB = F // _NF


def _ffn_body(x_ref, w1_ref, w2_ref, sc_ref, y_ref):
    x = x_ref[...]
    h = jnp.dot(x, w1_ref[0], preferred_element_type=jnp.float32)
    c0 = 0.7978845608028654
    g = 0.5 * h * (1.0 + jnp.tanh(c0 * (h + 0.044715 * h * h * h)))
    y_ref[...] = jnp.dot(
        g, w2_ref[0], preferred_element_type=jnp.float32) * sc_ref[:, 0:1]


def _ffn(buf, scale, W1, W2):
    return pl.pallas_call(
        _ffn_body,
        grid=(E, _NF),
        in_specs=[
            pl.BlockSpec((CP, D), lambda e, f: (e, 0)),
            pl.BlockSpec((1, D, _FB), lambda e, f: (e, 0, f)),
            pl.BlockSpec((1, _FB, D), lambda e, f: (e, f, 0)),
            pl.BlockSpec((CP, SW), lambda e, f: (e, 0)),
        ],
        out_specs=pl.BlockSpec((CP, D), lambda e, f: (e, 0)),
        out_shape=jax.ShapeDtypeStruct((BR, D), jnp.float32),
    )(buf, W1, W2, scale)


_CCH = 16


def _combine_body(y, cmb0, cmb1, out,
                  idx0a, idx1a, idx0b, idx1b, r0a, r1a, r0b, r1b,
                  sema, semb):
    c = lax.axis_index("c")
    s = lax.axis_index("s")
    w = s * 2 + c
    t0 = w * TPT

    bufs = ((idx0a, idx1a, r0a, r1a, sema), (idx0b, idx1b, r0b, r1b, semb))
    nch = TPT // _CCH

    def issue(ch, bset):
        i0, i1, r0, r1, sem = bset
        base = t0 + ch * _CCH
        pltpu.sync_copy(cmb0.at[pl.ds(base, _CCH)], i0)
        pltpu.sync_copy(cmb1.at[pl.ds(base, _CCH)], i1)
        return (pltpu.async_copy(y.at[i0], r0, sem),
                pltpu.async_copy(y.at[i1], r1, sem))

    pend = issue(0, bufs[0])
    for ch in range(nch):
        cur = bufs[ch % 2]
        d0, d1 = pend
        if ch + 1 < nch:
            pend = issue(ch + 1, bufs[(ch + 1) % 2])
        d0.wait()
        d1.wait()
        _, _, r0, r1, _ = cur

        def vadd(v, _):
            for tk in range(_CCH):
                r0[tk, pl.ds(v * 16, 16)] = (
                    r0[tk, pl.ds(v * 16, 16)] + r1[tk, pl.ds(v * 16, 16)])
            return 0
        lax.fori_loop(0, D // 16, vadd, 0)
        pltpu.sync_copy(r0, out.at[pl.ds(t0 + ch * _CCH, _CCH)])


def _combine(y, cmb0, cmb1):
    mesh = plsc.VectorSubcoreMesh(core_axis_name="c", subcore_axis_name="s")
    f32 = jnp.float32
    i32 = jnp.int32
    kfn = functools.partial(
        pl.kernel,
        out_type=jax.ShapeDtypeStruct((T, D), f32),
        mesh=mesh,
        scratch_types=[
            pltpu.VMEM((_CCH,), i32),
            pltpu.VMEM((_CCH,), i32),
            pltpu.VMEM((_CCH,), i32),
            pltpu.VMEM((_CCH,), i32),
            pltpu.VMEM((_CCH, D), f32),
            pltpu.VMEM((_CCH, D), f32),
            pltpu.VMEM((_CCH, D), f32),
            pltpu.VMEM((_CCH, D), f32),
            pltpu.SemaphoreType.DMA,
            pltpu.SemaphoreType.DMA,
        ],
    )(_combine_body)
    return kfn(y, cmb0, cmb1)


def kernel(hidden_states, router_logits, W1, W2):
    dsp0, dsp1, cmb0, cmb1, wb0, wb1 = _router(router_logits)
    buf, scale = _dispatch(
        hidden_states, dsp0.reshape(T), dsp1.reshape(T), wb0, wb1)
    y = _ffn(buf, scale, W1, W2)
    return _combine(y, cmb0.reshape(T), cmb1.reshape(T))
